# Initial kernel scaffold; baseline (speedup 1.0000x reference)
import jax
import jax.numpy as jnp
from jax.experimental import pallas as pl


def kernel(x, input_embedding_table):
    raise NotImplementedError("write your pallas kernel here")


# speedup vs baseline: 2.4119x; 2.4119x over previous
import functools

import jax
import jax.numpy as jnp
import numpy as np
from jax import lax
from jax.experimental import pallas as pl
from jax.experimental.pallas import tpu as pltpu
from jax.experimental.pallas import tpu_sc as plsc

VOCAB = 100000
D = 128
BATCH = 4096
HIST = 50
N = BATCH * HIST

_info = plsc.get_sparse_core_info()
NC = _info.num_cores
NS = _info.num_subcores
NW = NC * NS
LANES = 16

BPW = N // NW
CHUNK = 128
NCHUNK = BPW // CHUNK

SCALE = float(np.sqrt(np.float32(D)))


def _gather_body(idx_hbm, table_hbm, out_hbm, idx_v, rows_v, sems):
    wid = lax.axis_index("s") * NC + lax.axis_index("c")
    base = wid * BPW

    pltpu.sync_copy(idx_hbm.at[wid], idx_v)

    def chunk_body(c, _):
        pltpu.async_copy(table_hbm.at[idx_v.at[c]], rows_v, sems).wait()

        def row_body(r, _):
            for j in range(D // LANES):
                sl = pl.ds(j * LANES, LANES)
                rows_v[r, sl] = rows_v[r, sl] * SCALE
            return 0

        lax.fori_loop(0, CHUNK, row_body, 0)

        pltpu.sync_copy(rows_v, out_hbm.at[pl.ds(base + c * CHUNK, CHUNK)])
        return 0

    lax.fori_loop(0, NCHUNK, chunk_body, 0)


@jax.jit
def _embed(x_flat, table):
    idx = x_flat.reshape(NW, NCHUNK, CHUNK)
    call = functools.partial(
        pl.kernel,
        mesh=plsc.VectorSubcoreMesh(core_axis_name="c", subcore_axis_name="s"),
        out_type=jax.ShapeDtypeStruct((N, D), jnp.float32),
        scratch_types=[
            pltpu.VMEM((NCHUNK, CHUNK), jnp.int32),
            pltpu.VMEM((CHUNK, D), jnp.float32),
            pltpu.SemaphoreType.DMA,
        ],
    )(_gather_body)
    return call(idx, table)


def kernel(x, input_embedding_table):
    x_flat = x.reshape(-1).astype(jnp.int32)
    out = _embed(x_flat, input_embedding_table)
    return out.reshape(BATCH, HIST, D)


# speedup vs baseline: 2.9283x; 1.2141x over previous
import functools

import jax
import jax.numpy as jnp
import numpy as np
from jax import lax
from jax.experimental import pallas as pl
from jax.experimental.pallas import tpu as pltpu
from jax.experimental.pallas import tpu_sc as plsc

VOCAB = 100000
D = 128
BATCH = 4096
HIST = 50
N = BATCH * HIST

_info = plsc.get_sparse_core_info()
NC = _info.num_cores
NS = _info.num_subcores
NW = NC * NS
LANES = 16

BPW = N // NW
CHUNK = 128
NCHUNK = BPW // CHUNK

SCALE = float(np.sqrt(np.float32(D)))


def _gather_body(idx_hbm, table_hbm, out_hbm,
                 idx_v, rin0, rin1, rout0, rout1,
                 gs0, gs1, os0, os1):
    wid = lax.axis_index("s") * NC + lax.axis_index("c")
    base = wid * BPW

    pltpu.sync_copy(idx_hbm.at[wid], idx_v)

    bufs = ((rin0, gs0, rout0, os0), (rin1, gs1, rout1, os1))

    def start_gather(c, rin, gsem):
        pltpu.async_copy(table_hbm.at[idx_v.at[c]], rin, gsem)

    def wait_gather(c, rin, gsem):
        pltpu.make_async_copy(table_hbm.at[idx_v.at[c]], rin, gsem).wait()

    def start_out(c, rout, osem):
        pltpu.async_copy(rout, out_hbm.at[pl.ds(base + c * CHUNK, CHUNK)], osem)

    def wait_out(c, rout, osem):
        pltpu.make_async_copy(
            rout, out_hbm.at[pl.ds(base + c * CHUNK, CHUNK)], osem).wait()

    def scale_chunk(rin, rout):
        def row_body(r, _):
            for j in range(D // LANES):
                sl = pl.ds(j * LANES, LANES)
                rout[r, sl] = rin[r, sl] * SCALE
            return 0

        lax.fori_loop(0, CHUNK, row_body, 0)

    start_gather(0, rin0, gs0)
    start_gather(1, rin1, gs1)
    for b in range(2):
        rin, gsem, rout, osem = bufs[b]
        wait_gather(b, rin, gsem)
        scale_chunk(rin, rout)
        start_out(b, rout, osem)
        start_gather(b + 2, rin, gsem)

    def steady(g, _):
        for b in range(2):
            c = 2 * g + b
            rin, gsem, rout, osem = bufs[b]
            wait_gather(c, rin, gsem)
            wait_out(c - 2, rout, osem)
            scale_chunk(rin, rout)
            start_out(c, rout, osem)
            start_gather(c + 2, rin, gsem)
        return 0

    lax.fori_loop(1, NCHUNK // 2 - 1, steady, 0)

    for b in range(2):
        c = NCHUNK - 2 + b
        rin, gsem, rout, osem = bufs[b]
        wait_gather(c, rin, gsem)
        wait_out(c - 2, rout, osem)
        scale_chunk(rin, rout)
        start_out(c, rout, osem)
    for b in range(2):
        c = NCHUNK - 2 + b
        rin, gsem, rout, osem = bufs[b]
        wait_out(c, rout, osem)


@jax.jit
def _embed(x_flat, table):
    idx = x_flat.reshape(NW, NCHUNK, CHUNK)
    call = functools.partial(
        pl.kernel,
        mesh=plsc.VectorSubcoreMesh(core_axis_name="c", subcore_axis_name="s"),
        out_type=jax.ShapeDtypeStruct((N, D), jnp.float32),
        scratch_types=[
            pltpu.VMEM((NCHUNK, CHUNK), jnp.int32),
            pltpu.VMEM((CHUNK, D), jnp.float32),
            pltpu.VMEM((CHUNK, D), jnp.float32),
            pltpu.VMEM((CHUNK, D), jnp.float32),
            pltpu.VMEM((CHUNK, D), jnp.float32),
            pltpu.SemaphoreType.DMA,
            pltpu.SemaphoreType.DMA,
            pltpu.SemaphoreType.DMA,
            pltpu.SemaphoreType.DMA,
        ],
    )(_gather_body)
    return call(idx, table)


def kernel(x, input_embedding_table):
    x_flat = x.reshape(-1).astype(jnp.int32)
    out = _embed(x_flat, input_embedding_table)
    return out.reshape(BATCH, HIST, D)


# speedup vs baseline: 4.6135x; 1.5755x over previous
import functools

import jax
import jax.numpy as jnp
import numpy as np
from jax import lax
from jax.experimental import pallas as pl
from jax.experimental.pallas import tpu as pltpu
from jax.experimental.pallas import tpu_sc as plsc

VOCAB = 100000
D = 128
BATCH = 4096
HIST = 50

_info = plsc.get_sparse_core_info()
NC = _info.num_cores
NS = _info.num_subcores
NW = NC * NS
LANES = 16

BPW = BATCH // NW

SCALE = float(np.sqrt(np.float32(D)))


def _gather_body(idx_hbm, table_hbm, out_hbm,
                 idx_v, rin0, rin1, rout0, rout1,
                 gs0, gs1, os0, os1):
    wid = lax.axis_index("s") * NC + lax.axis_index("c")
    base = wid * BPW

    pltpu.sync_copy(idx_hbm.at[wid], idx_v)

    bufs = ((rin0, gs0, rout0, os0), (rin1, gs1, rout1, os1))

    def start_gather(c, rin, gsem):
        pltpu.async_copy(table_hbm.at[idx_v.at[c]], rin, gsem)

    def wait_gather(c, rin, gsem):
        pltpu.make_async_copy(table_hbm.at[idx_v.at[c]], rin, gsem).wait()

    def start_out(c, rout, osem):
        pltpu.async_copy(rout, out_hbm.at[base + c], osem)

    def wait_out(c, rout, osem):
        pltpu.make_async_copy(rout, out_hbm.at[base + c], osem).wait()

    def scale_chunk(rin, rout):
        def row_body(r, _):
            for j in range(D // LANES):
                sl = pl.ds(j * LANES, LANES)
                rout[r, sl] = rin[r, sl] * SCALE
            return 0

        lax.fori_loop(0, HIST, row_body, 0)

    start_gather(0, rin0, gs0)
    start_gather(1, rin1, gs1)
    for b in range(2):
        rin, gsem, rout, osem = bufs[b]
        wait_gather(b, rin, gsem)
        scale_chunk(rin, rout)
        start_out(b, rout, osem)
        start_gather(b + 2, rin, gsem)

    def steady(g, _):
        for b in range(2):
            c = 2 * g + b
            rin, gsem, rout, osem = bufs[b]
            wait_gather(c, rin, gsem)
            wait_out(c - 2, rout, osem)
            scale_chunk(rin, rout)
            start_out(c, rout, osem)
            start_gather(c + 2, rin, gsem)
        return 0

    lax.fori_loop(1, BPW // 2 - 1, steady, 0)

    for b in range(2):
        c = BPW - 2 + b
        rin, gsem, rout, osem = bufs[b]
        wait_gather(c, rin, gsem)
        wait_out(c - 2, rout, osem)
        scale_chunk(rin, rout)
        start_out(c, rout, osem)
    for b in range(2):
        c = BPW - 2 + b
        rin, gsem, rout, osem = bufs[b]
        wait_out(c, rout, osem)


@jax.jit
def _embed(x, table):
    idx = x.reshape(NW, BPW, HIST)
    call = functools.partial(
        pl.kernel,
        mesh=plsc.VectorSubcoreMesh(core_axis_name="c", subcore_axis_name="s"),
        out_type=jax.ShapeDtypeStruct((BATCH, HIST, D), jnp.float32),
        scratch_types=[
            pltpu.VMEM((BPW, HIST), jnp.int32),
            pltpu.VMEM((HIST, D), jnp.float32),
            pltpu.VMEM((HIST, D), jnp.float32),
            pltpu.VMEM((HIST, D), jnp.float32),
            pltpu.VMEM((HIST, D), jnp.float32),
            pltpu.SemaphoreType.DMA,
            pltpu.SemaphoreType.DMA,
            pltpu.SemaphoreType.DMA,
            pltpu.SemaphoreType.DMA,
        ],
    )(_gather_body)
    return call(idx, table)


def kernel(x, input_embedding_table):
    return _embed(x.astype(jnp.int32), input_embedding_table)


# speedup vs baseline: 4.6142x; 1.0002x over previous
import functools

import jax
import jax.numpy as jnp
import numpy as np
from jax import lax
from jax.experimental import pallas as pl
from jax.experimental.pallas import tpu as pltpu
from jax.experimental.pallas import tpu_sc as plsc

VOCAB = 100000
D = 128
BATCH = 4096
HIST = 50

_info = plsc.get_sparse_core_info()
NC = _info.num_cores
NS = _info.num_subcores
NW = NC * NS
LANES = 16

BPW = BATCH // NW

SCALE = float(np.sqrt(np.float32(D)))


def _gather_body(idx_hbm, table_hbm, out_hbm,
                 idx_v, rin0, rin1, rout0, rout1,
                 gs0, gs1, os0, os1):
    wid = lax.axis_index("s") * NC + lax.axis_index("c")
    base = wid * BPW

    pltpu.sync_copy(idx_hbm.at[wid], idx_v)

    bufs = ((rin0, gs0, rout0, os0), (rin1, gs1, rout1, os1))

    def start_gather(c, rin, gsem):
        pltpu.async_copy(table_hbm.at[idx_v.at[c]], rin, gsem)

    def wait_gather(c, rin, gsem):
        pltpu.make_async_copy(table_hbm.at[idx_v.at[c]], rin, gsem).wait()

    def start_out(c, rout, osem):
        pltpu.async_copy(rout, out_hbm.at[base + c], osem)

    def wait_out(c, rout, osem):
        pltpu.make_async_copy(rout, out_hbm.at[base + c], osem).wait()

    def scale_chunk(rin, rout):
        def row_body(r, _):
            for j in range(D // LANES):
                sl = pl.ds(j * LANES, LANES)
                rout[r, sl] = rin[r, sl] * SCALE
            return 0

        lax.fori_loop(0, HIST, row_body, 0)

    start_gather(0, rin0, gs0)
    start_gather(1, rin1, gs1)
    for b in range(2):
        rin, gsem, rout, osem = bufs[b]
        wait_gather(b, rin, gsem)
        scale_chunk(rin, rout)
        start_out(b, rout, osem)
        start_gather(b + 2, rin, gsem)

    def steady(g, _):
        for b in range(2):
            c = 2 * g + b
            rin, gsem, rout, osem = bufs[b]
            wait_gather(c, rin, gsem)
            wait_out(c - 2, rout, osem)
            scale_chunk(rin, rout)
            start_out(c, rout, osem)
            start_gather(c + 2, rin, gsem)
        return 0

    lax.fori_loop(1, BPW // 2 - 1, steady, 0)

    for b in range(2):
        c = BPW - 2 + b
        rin, gsem, rout, osem = bufs[b]
        wait_gather(c, rin, gsem)
        wait_out(c - 2, rout, osem)
        scale_chunk(rin, rout)
        start_out(c, rout, osem)
    for b in range(2):
        c = BPW - 2 + b
        rin, gsem, rout, osem = bufs[b]
        wait_out(c, rout, osem)


@jax.jit
def _embed(x, table):
    idx = x.reshape(NW, BPW, HIST)
    call = functools.partial(
        pl.kernel,
        mesh=plsc.VectorSubcoreMesh(core_axis_name="c", subcore_axis_name="s"),
        compiler_params=pltpu.CompilerParams(use_tc_tiling_on_sc=True),
        out_type=jax.ShapeDtypeStruct((BATCH, HIST, D), jnp.float32),
        scratch_types=[
            pltpu.VMEM((BPW, HIST), jnp.int32),
            pltpu.VMEM((HIST, D), jnp.float32),
            pltpu.VMEM((HIST, D), jnp.float32),
            pltpu.VMEM((HIST, D), jnp.float32),
            pltpu.VMEM((HIST, D), jnp.float32),
            pltpu.SemaphoreType.DMA,
            pltpu.SemaphoreType.DMA,
            pltpu.SemaphoreType.DMA,
            pltpu.SemaphoreType.DMA,
        ],
    )(_gather_body)
    return call(idx, table)


def kernel(x, input_embedding_table):
    return _embed(x.astype(jnp.int32), input_embedding_table)


# speedup vs baseline: 5.1405x; 1.1141x over previous
import functools

import jax
import jax.numpy as jnp
import numpy as np
from jax import lax
from jax.experimental import pallas as pl
from jax.experimental.pallas import tpu as pltpu
from jax.experimental.pallas import tpu_sc as plsc

VOCAB = 100000
D = 128
BATCH = 4096
HIST = 50

_info = plsc.get_sparse_core_info()
NC = _info.num_cores
NS = _info.num_subcores
NW = NC * NS
LANES = 16

BPW = BATCH // NW
BPC = 2
CIDX = BPC * HIST
NCHUNK = BPW // BPC

SCALE = float(np.sqrt(np.float32(D)))


def _gather_body(idx_hbm, table_hbm, out_hbm,
                 idx_v, rin0, rin1, rout0, rout1,
                 gs0, gs1, os0, os1):
    wid = lax.axis_index("s") * NC + lax.axis_index("c")
    base = wid * BPW

    pltpu.sync_copy(idx_hbm.at[wid], idx_v)

    bufs = ((rin0, gs0, rout0, os0), (rin1, gs1, rout1, os1))

    def start_gather(c, rin, gsem):
        pltpu.async_copy(table_hbm.at[idx_v.at[c]], rin, gsem)

    def wait_gather(c, rin, gsem):
        pltpu.make_async_copy(table_hbm.at[idx_v.at[c]], rin, gsem).wait()

    def start_out(c, rout, osem):
        pltpu.async_copy(rout, out_hbm.at[pl.ds(base + c * BPC, BPC)], osem)

    def wait_out(c, rout, osem):
        pltpu.make_async_copy(
            rout, out_hbm.at[pl.ds(base + c * BPC, BPC)], osem).wait()

    def scale_chunk(rin, rout):
        for bb in range(BPC):
            def row_body(h, _, bb=bb):
                for j in range(D // LANES):
                    sl = pl.ds(j * LANES, LANES)
                    rout[bb, h, sl] = rin[bb * HIST + h, sl] * SCALE
                return 0

            lax.fori_loop(0, HIST, row_body, 0)

    start_gather(0, rin0, gs0)
    start_gather(1, rin1, gs1)
    for b in range(2):
        rin, gsem, rout, osem = bufs[b]
        wait_gather(b, rin, gsem)
        scale_chunk(rin, rout)
        start_out(b, rout, osem)
        start_gather(b + 2, rin, gsem)

    def steady(g, _):
        for b in range(2):
            c = 2 * g + b
            rin, gsem, rout, osem = bufs[b]
            wait_gather(c, rin, gsem)
            wait_out(c - 2, rout, osem)
            scale_chunk(rin, rout)
            start_out(c, rout, osem)
            start_gather(c + 2, rin, gsem)
        return 0

    lax.fori_loop(1, NCHUNK // 2 - 1, steady, 0)

    for b in range(2):
        c = NCHUNK - 2 + b
        rin, gsem, rout, osem = bufs[b]
        wait_gather(c, rin, gsem)
        wait_out(c - 2, rout, osem)
        scale_chunk(rin, rout)
        start_out(c, rout, osem)
    for b in range(2):
        c = NCHUNK - 2 + b
        rin, gsem, rout, osem = bufs[b]
        wait_out(c, rout, osem)


@jax.jit
def _embed(x, table):
    idx = x.reshape(NW, NCHUNK, CIDX)
    call = functools.partial(
        pl.kernel,
        mesh=plsc.VectorSubcoreMesh(core_axis_name="c", subcore_axis_name="s"),
        out_type=jax.ShapeDtypeStruct((BATCH, HIST, D), jnp.float32),
        scratch_types=[
            pltpu.VMEM((NCHUNK, CIDX), jnp.int32),
            pltpu.VMEM((CIDX, D), jnp.float32),
            pltpu.VMEM((CIDX, D), jnp.float32),
            pltpu.VMEM((BPC, HIST, D), jnp.float32),
            pltpu.VMEM((BPC, HIST, D), jnp.float32),
            pltpu.SemaphoreType.DMA,
            pltpu.SemaphoreType.DMA,
            pltpu.SemaphoreType.DMA,
            pltpu.SemaphoreType.DMA,
        ],
    )(_gather_body)
    return call(idx, table)


def kernel(x, input_embedding_table):
    return _embed(x.astype(jnp.int32), input_embedding_table)
